# baseline (speedup 1.0000x reference)
import jax
import jax.numpy as jnp
from jax import lax
from jax.experimental import pallas as pl
from jax.experimental.pallas import tpu as pltpu
from jax.experimental.pallas import tpu_sc as plsc

N = 10000
E = 320000
F_IN = 128
F_OUT = 32
F3 = 3 * F_OUT
NC = 2
NS = 16
NW = NC * NS

EPW_A = E // NW


def _deg_body(col_hbm, w_hbm, degp_hbm, col_v, w_v, acc_v):
    wid = lax.axis_index("s") * NC + lax.axis_index("c")
    base = wid * EPW_A
    pltpu.sync_copy(col_hbm.at[pl.ds(base, EPW_A)], col_v)
    pltpu.sync_copy(w_hbm.at[pl.ds(base, EPW_A)], w_v)

    zero16 = jnp.zeros((16,), jnp.float32)

    @plsc.parallel_loop(0, N // 16)
    def _zero(j):
        acc_v[pl.ds(j * 16, 16)] = zero16

    @plsc.parallel_loop(0, EPW_A // 16, unroll=4)
    def _edges(g):
        b = g * 16
        c16 = col_v[pl.ds(b, 16)]
        w16 = w_v[pl.ds(b, 16)]
        plsc.addupdate_scatter(acc_v, [c16], w16)

    pltpu.sync_copy(acc_v, degp_hbm.at[wid])


def _deg_partials(col, w):
    mesh = plsc.VectorSubcoreMesh(core_axis_name="c", subcore_axis_name="s")
    return pl.kernel(
        _deg_body,
        compiler_params=pltpu.CompilerParams(needs_layout_passes=False, use_tc_tiling_on_sc=False),
        out_type=jax.ShapeDtypeStruct((NW, N), jnp.float32),
        mesh=mesh,
        scratch_types=[
            pltpu.VMEM((EPW_A,), jnp.int32),
            pltpu.VMEM((EPW_A,), jnp.float32),
            pltpu.VMEM((N,), jnp.float32),
        ],
    )(col, w)


BN_B = 500


def _xwn_body(x_ref, degp_ref, wcatT_ref, xwnT_ref, xwnP_ref, dis_ref):
    deg = jnp.sum(degp_ref[...], axis=0, keepdims=True) + 1.0
    dis = lax.rsqrt(deg)
    xwT = lax.dot_general(wcatT_ref[...], x_ref[...],
                          (((1,), (1,)), ((), ())),
                          preferred_element_type=jnp.float32)
    xwn = xwT * dis
    xwnT_ref[...] = xwn
    x3 = xwn.reshape(SPLIT_F, FPW, N)
    u_lo = lax.bitcast_convert_type(
        x3[:, 0:FPW // 2].astype(jnp.bfloat16), jnp.uint16).astype(jnp.uint32)
    u_hi = lax.bitcast_convert_type(
        x3[:, FPW // 2:FPW].astype(jnp.bfloat16), jnp.uint16).astype(jnp.uint32)
    word = u_lo | (u_hi << 16)
    xwnP_ref[...] = lax.bitcast_convert_type(word, jnp.int32).reshape(F3 // 2, N)
    dis_ref[...] = dis


def _xwn(x, degp, wcatT):
    return pl.pallas_call(
        _xwn_body,
        out_shape=[
            jax.ShapeDtypeStruct((F3, N), jnp.float32),
            jax.ShapeDtypeStruct((F3 // 2, N), jnp.int32),
            jax.ShapeDtypeStruct((1, N), jnp.float32),
        ],
    )(x, degp, wcatT)


SPLIT_F = 16
SPLIT_E = NW // SPLIT_F
FPW = F3 // SPLIT_F
PPW = FPW // 2
EPW_C = E // SPLIT_E
CH_C = 4000
NCH_C = EPW_C // CH_C
NCH_T = E // CH_C
NBUF = 2
MASK_HI = jnp.int32(-65536)
MASK_LO = jnp.int32(0xFFFF)


def _msg_body(epack_hbm, xwnP_hbm, accp_hbm, table_v, acc_v, e_v, sems):
    wid = lax.axis_index("s") * NC + lax.axis_index("c")
    fgroup = wid % SPLIT_F
    eslice = wid // SPLIT_F
    cbase = eslice * NCH_C

    def issue(ci, b):
        cidx = cbase + jnp.minimum(ci, NCH_C - 1)
        pltpu.async_copy(epack_hbm.at[cidx], e_v.at[b], sems.at[b])

    def drain(b):
        pltpu.make_async_copy(epack_hbm.at[0], e_v.at[b], sems.at[b]).wait()

    for b in range(NBUF):
        issue(jnp.int32(b), b)

    pltpu.sync_copy(xwnP_hbm.at[pl.ds(fgroup * PPW, PPW)], table_v)

    zero16 = jnp.zeros((16,), jnp.float32)
    for f in range(FPW):
        @plsc.parallel_loop(0, N // 16)
        def _zero(j, f=f):
            acc_v[f, pl.ds(j * 16, 16)] = zero16

    @pl.loop(0, NCH_C, step=NBUF)
    def _chunk(g):
        for b in range(NBUF):
            ci = g + b
            drain(b)

            @plsc.parallel_loop(0, CH_C // 16, unroll=4)
            def _group(gg, b=b):
                o = gg * 16
                rc = e_v[b, 0, pl.ds(o, 16)]
                r16 = rc & MASK_LO
                c16 = lax.shift_right_logical(rc, 16)
                w16 = plsc.bitcast(e_v[b, 1, pl.ds(o, 16)], jnp.float32)
                for g2 in range(PPW):
                    t2 = plsc.load_gather(table_v.at[g2], [r16])
                    tlo = plsc.bitcast(lax.shift_left(t2, 16), jnp.float32)
                    thi = plsc.bitcast(t2 & MASK_HI, jnp.float32)
                    plsc.addupdate_scatter(acc_v.at[g2], [c16], tlo * w16)
                    plsc.addupdate_scatter(acc_v.at[g2 + PPW], [c16], thi * w16)

            issue(ci + NBUF, b)

    for b in range(NBUF):
        drain(b)
    pltpu.sync_copy(acc_v, accp_hbm.at[eslice, pl.ds(fgroup * FPW, FPW)])


def _msg_partials(epack, xwnP):
    mesh = plsc.VectorSubcoreMesh(core_axis_name="c", subcore_axis_name="s")
    return pl.kernel(
        _msg_body,
        compiler_params=pltpu.CompilerParams(needs_layout_passes=False, use_tc_tiling_on_sc=False),
        out_type=jax.ShapeDtypeStruct((SPLIT_E, F3, N), jnp.float32),
        mesh=mesh,
        scratch_types=[
            pltpu.VMEM((PPW, N), jnp.int32),
            pltpu.VMEM((FPW, N), jnp.float32),
            pltpu.VMEM((NBUF, 2, CH_C), jnp.int32),
            pltpu.SemaphoreType.DMA((NBUF,)),
        ],
    )(epack, xwnP)


BN_D = 500


def _gates_body(accp_ref, xwnT_ref, dis_ref, hT_ref, bcat_ref,
                wz1_ref, wz2_ref, bz_ref, wr1_ref, wr2_ref, br_ref,
                wh1_ref, wh2_ref, bh_ref, wo_ref, bo_ref,
                hnT_ref, yT_ref):
    acc = accp_ref[0] + accp_ref[1]
    dis = dis_ref[...]
    S = dis * (acc + xwnT_ref[...]) + bcat_ref[...]
    cz = S[0:F_OUT]
    cr = S[F_OUT:2 * F_OUT]
    ch = S[2 * F_OUT:F3]
    H = hT_ref[...]

    def mm(a, b):
        return jnp.dot(a, b, preferred_element_type=jnp.float32)

    Z = jax.nn.sigmoid(mm(wz1_ref[...], cz) + mm(wz2_ref[...], H) + bz_ref[...])
    R = jax.nn.sigmoid(mm(wr1_ref[...], cr) + mm(wr2_ref[...], H) + br_ref[...])
    Ht = jnp.tanh(mm(wh1_ref[...], ch) + mm(wh2_ref[...], H * R) + bh_ref[...])
    Hn = Z * H + (1.0 - Z) * Ht
    hnT_ref[...] = Hn
    yT_ref[...] = mm(wo_ref[...], jnp.maximum(Hn, 0.0)) + bo_ref[...]


def _gates(accp, xwnT, dis, hT, bcat, wz1, wz2, bz, wr1, wr2, br,
           wh1, wh2, bh, wo, bo):
    return pl.pallas_call(
        _gates_body,
        out_shape=[
            jax.ShapeDtypeStruct((F_OUT, N), jnp.float32),
            jax.ShapeDtypeStruct((1, N), jnp.float32),
        ],
    )(accp, xwnT, dis, hT, bcat, wz1, wz2, bz, wr1, wr2, br,
      wh1, wh2, bh, wo, bo)


def kernel(x, edge_index, edge_weight, prev_hidden_state,
           W_conv_z, b_conv_z, W_conv_r, b_conv_r, W_conv_h, b_conv_h,
           W_lin_z, b_lin_z, W_lin_r, b_lin_r, W_lin_h, b_lin_h,
           W_out, b_out):
    row = edge_index[0]
    col = edge_index[1]

    wbits = lax.bitcast_convert_type(edge_weight, jnp.int32)
    rc = row | (col << 16)
    epack = jnp.stack([rc.reshape(NCH_T, CH_C),
                       wbits.reshape(NCH_T, CH_C)], axis=1)

    hT = prev_hidden_state.T
    wcatT = jnp.concatenate([W_conv_z, W_conv_r, W_conv_h], axis=1).T
    bcat = jnp.concatenate([b_conv_z, b_conv_r, b_conv_h]).reshape(F3, 1)

    degp = _deg_partials(col, edge_weight)
    xwnT, xwnP, dis = _xwn(x, degp, wcatT)
    accp = _msg_partials(epack, xwnP)

    hnT, yT = _gates(
        accp, xwnT, dis, hT, bcat,
        W_lin_z[:F_OUT].T, W_lin_z[F_OUT:].T, b_lin_z.reshape(F_OUT, 1),
        W_lin_r[:F_OUT].T, W_lin_r[F_OUT:].T, b_lin_r.reshape(F_OUT, 1),
        W_lin_h[:F_OUT].T, W_lin_h[F_OUT:].T, b_lin_h.reshape(F_OUT, 1),
        W_out.T, b_out.reshape(1, 1),
    )
    return yT.T, hnT.T


# speedup vs baseline: 1.0127x; 1.0127x over previous
import jax
import jax.numpy as jnp
from jax import lax
from jax.experimental import pallas as pl
from jax.experimental.pallas import tpu as pltpu
from jax.experimental.pallas import tpu_sc as plsc

N = 10000
E = 320000
F_IN = 128
F_OUT = 32
F3 = 3 * F_OUT
NC = 2
NS = 16
NW = NC * NS

EPW_A = E // NW


def _deg_body(col_hbm, w_hbm, degp_hbm, col_v, w_v, acc_v):
    wid = lax.axis_index("s") * NC + lax.axis_index("c")
    base = wid * EPW_A
    pltpu.sync_copy(col_hbm.at[pl.ds(base, EPW_A)], col_v)
    pltpu.sync_copy(w_hbm.at[pl.ds(base, EPW_A)], w_v)

    zero16 = jnp.zeros((16,), jnp.float32)

    @plsc.parallel_loop(0, N // 16)
    def _zero(j):
        acc_v[pl.ds(j * 16, 16)] = zero16

    @plsc.parallel_loop(0, EPW_A // 16, unroll=4)
    def _edges(g):
        b = g * 16
        c16 = col_v[pl.ds(b, 16)]
        w16 = w_v[pl.ds(b, 16)]
        plsc.addupdate_scatter(acc_v, [c16], w16)

    pltpu.sync_copy(acc_v, degp_hbm.at[wid])


def _deg_partials(col, w):
    mesh = plsc.VectorSubcoreMesh(core_axis_name="c", subcore_axis_name="s")
    return pl.kernel(
        _deg_body,
        compiler_params=pltpu.CompilerParams(needs_layout_passes=False, use_tc_tiling_on_sc=False),
        out_type=jax.ShapeDtypeStruct((NW, N), jnp.float32),
        mesh=mesh,
        scratch_types=[
            pltpu.VMEM((EPW_A,), jnp.int32),
            pltpu.VMEM((EPW_A,), jnp.float32),
            pltpu.VMEM((N,), jnp.float32),
        ],
    )(col, w)


BN_B = 500


def _xwn_body(x_ref, degp_ref, wcatT_ref, xwnT_ref, xwnP_ref, dis_ref):
    deg = jnp.sum(degp_ref[...], axis=0, keepdims=True) + 1.0
    dis = lax.rsqrt(deg)
    xwT = lax.dot_general(wcatT_ref[...], x_ref[...],
                          (((1,), (1,)), ((), ())),
                          preferred_element_type=jnp.float32)
    xwn = xwT * dis
    xwnT_ref[...] = xwn
    x3 = xwn.reshape(SPLIT_F, FPW, N)
    u_lo = lax.bitcast_convert_type(
        x3[:, 0:FPW // 2].astype(jnp.bfloat16), jnp.uint16).astype(jnp.uint32)
    u_hi = lax.bitcast_convert_type(
        x3[:, FPW // 2:FPW].astype(jnp.bfloat16), jnp.uint16).astype(jnp.uint32)
    word = u_lo | (u_hi << 16)
    xwnP_ref[...] = lax.bitcast_convert_type(word, jnp.int32).reshape(F3 // 2, N)
    dis_ref[...] = dis


def _xwn(x, degp, wcatT):
    return pl.pallas_call(
        _xwn_body,
        out_shape=[
            jax.ShapeDtypeStruct((F3, N), jnp.float32),
            jax.ShapeDtypeStruct((F3 // 2, N), jnp.int32),
            jax.ShapeDtypeStruct((1, N), jnp.float32),
        ],
    )(x, degp, wcatT)


SPLIT_F = 16
SPLIT_E = NW // SPLIT_F
FPW = F3 // SPLIT_F
PPW = FPW // 2
EPW_C = E // SPLIT_E
CH_C = 4000
NCH_C = EPW_C // CH_C
NCH_T = E // CH_C
NBUF = 2
MASK_HI = jnp.int32(-65536)
MASK_LO = jnp.int32(0xFFFF)


def _msg_body(epack_hbm, xwnP_hbm, accp_hbm, table_v, acc_v, e_v, sems):
    wid = lax.axis_index("s") * NC + lax.axis_index("c")
    fgroup = wid % SPLIT_F
    eslice = wid // SPLIT_F
    cbase = eslice * NCH_C

    def issue(ci, b):
        cidx = cbase + jnp.minimum(ci, NCH_C - 1)
        pltpu.async_copy(epack_hbm.at[cidx], e_v.at[b], sems.at[b])

    def drain(b):
        pltpu.make_async_copy(epack_hbm.at[0], e_v.at[b], sems.at[b]).wait()

    for b in range(NBUF):
        issue(jnp.int32(b), b)

    pltpu.sync_copy(xwnP_hbm.at[pl.ds(fgroup * PPW, PPW)], table_v)

    zero16 = jnp.zeros((16,), jnp.float32)
    for f in range(FPW):
        @plsc.parallel_loop(0, N // 16)
        def _zero(j, f=f):
            acc_v[f, pl.ds(j * 16, 16)] = zero16

    @pl.loop(0, NCH_C, step=NBUF)
    def _chunk(g):
        for b in range(NBUF):
            ci = g + b
            drain(b)

            @plsc.parallel_loop(0, CH_C // 16, unroll=2)
            def _group(gg, b=b):
                o = gg * 16
                rc = e_v[b, 0, pl.ds(o, 16)]
                r16 = rc & MASK_LO
                c16 = lax.shift_right_logical(rc, 16)
                w16 = plsc.bitcast(e_v[b, 1, pl.ds(o, 16)], jnp.float32)
                for g2 in range(PPW):
                    t2 = plsc.load_gather(table_v.at[g2], [r16])
                    tlo = plsc.bitcast(lax.shift_left(t2, 16), jnp.float32)
                    thi = plsc.bitcast(t2 & MASK_HI, jnp.float32)
                    plsc.addupdate_scatter(acc_v.at[g2], [c16], tlo * w16)
                    plsc.addupdate_scatter(acc_v.at[g2 + PPW], [c16], thi * w16)

            issue(ci + NBUF, b)

    for b in range(NBUF):
        drain(b)
    pltpu.sync_copy(acc_v, accp_hbm.at[eslice, pl.ds(fgroup * FPW, FPW)])


def _msg_partials(epack, xwnP):
    mesh = plsc.VectorSubcoreMesh(core_axis_name="c", subcore_axis_name="s")
    return pl.kernel(
        _msg_body,
        compiler_params=pltpu.CompilerParams(needs_layout_passes=False, use_tc_tiling_on_sc=False),
        out_type=jax.ShapeDtypeStruct((SPLIT_E, F3, N), jnp.float32),
        mesh=mesh,
        scratch_types=[
            pltpu.VMEM((PPW, N), jnp.int32),
            pltpu.VMEM((FPW, N), jnp.float32),
            pltpu.VMEM((NBUF, 2, CH_C), jnp.int32),
            pltpu.SemaphoreType.DMA((NBUF,)),
        ],
    )(epack, xwnP)


BN_D = 500


def _gates_body(accp_ref, xwnT_ref, dis_ref, hT_ref, bcat_ref,
                wz1_ref, wz2_ref, bz_ref, wr1_ref, wr2_ref, br_ref,
                wh1_ref, wh2_ref, bh_ref, wo_ref, bo_ref,
                hnT_ref, yT_ref):
    acc = accp_ref[0] + accp_ref[1]
    dis = dis_ref[...]
    S = dis * (acc + xwnT_ref[...]) + bcat_ref[...]
    cz = S[0:F_OUT]
    cr = S[F_OUT:2 * F_OUT]
    ch = S[2 * F_OUT:F3]
    H = hT_ref[...]

    def mm(a, b):
        return jnp.dot(a, b, preferred_element_type=jnp.float32)

    Z = jax.nn.sigmoid(mm(wz1_ref[...], cz) + mm(wz2_ref[...], H) + bz_ref[...])
    R = jax.nn.sigmoid(mm(wr1_ref[...], cr) + mm(wr2_ref[...], H) + br_ref[...])
    Ht = jnp.tanh(mm(wh1_ref[...], ch) + mm(wh2_ref[...], H * R) + bh_ref[...])
    Hn = Z * H + (1.0 - Z) * Ht
    hnT_ref[...] = Hn
    yT_ref[...] = mm(wo_ref[...], jnp.maximum(Hn, 0.0)) + bo_ref[...]


def _gates(accp, xwnT, dis, hT, bcat, wz1, wz2, bz, wr1, wr2, br,
           wh1, wh2, bh, wo, bo):
    return pl.pallas_call(
        _gates_body,
        out_shape=[
            jax.ShapeDtypeStruct((F_OUT, N), jnp.float32),
            jax.ShapeDtypeStruct((1, N), jnp.float32),
        ],
    )(accp, xwnT, dis, hT, bcat, wz1, wz2, bz, wr1, wr2, br,
      wh1, wh2, bh, wo, bo)


def kernel(x, edge_index, edge_weight, prev_hidden_state,
           W_conv_z, b_conv_z, W_conv_r, b_conv_r, W_conv_h, b_conv_h,
           W_lin_z, b_lin_z, W_lin_r, b_lin_r, W_lin_h, b_lin_h,
           W_out, b_out):
    row = edge_index[0]
    col = edge_index[1]

    wbits = lax.bitcast_convert_type(edge_weight, jnp.int32)
    rc = row | (col << 16)
    epack = jnp.stack([rc.reshape(NCH_T, CH_C),
                       wbits.reshape(NCH_T, CH_C)], axis=1)

    hT = prev_hidden_state.T
    wcatT = jnp.concatenate([W_conv_z, W_conv_r, W_conv_h], axis=1).T
    bcat = jnp.concatenate([b_conv_z, b_conv_r, b_conv_h]).reshape(F3, 1)

    degp = _deg_partials(col, edge_weight)
    xwnT, xwnP, dis = _xwn(x, degp, wcatT)
    accp = _msg_partials(epack, xwnP)

    hnT, yT = _gates(
        accp, xwnT, dis, hT, bcat,
        W_lin_z[:F_OUT].T, W_lin_z[F_OUT:].T, b_lin_z.reshape(F_OUT, 1),
        W_lin_r[:F_OUT].T, W_lin_r[F_OUT:].T, b_lin_r.reshape(F_OUT, 1),
        W_lin_h[:F_OUT].T, W_lin_h[F_OUT:].T, b_lin_h.reshape(F_OUT, 1),
        W_out.T, b_out.reshape(1, 1),
    )
    return yT.T, hnT.T
